# baseline (speedup 1.0000x reference)
import functools

import jax
import jax.numpy as jnp
from jax import lax
from jax.experimental import pallas as pl
from jax.experimental.pallas import tpu as pltpu
from jax.experimental.pallas import tpu_sc as plsc

B, L, V, D, H1, C = 16384, 50, 100000, 128, 1024, 1000
SLOPE = 0.01

NC, NS = 2, 16
NW = NC * NS
NSPLIT = 2
BS = B // NSPLIT
SPW = BS // NW
X2_COLS = 2 * L
ROWS_PW = SPW // 2
NBUF = 2
CPW = ROWS_PW // 2


def _pool_body(half, x2_hbm, tab_hbm, out_hbm, idx_v, rows_bufs, outb, gsems,
               osems):
    del half
    wid = lax.axis_index("s") * NC + lax.axis_index("c")
    row0 = wid * ROWS_PW
    out0 = wid * SPW

    pltpu.sync_copy(x2_hbm.at[pl.ds(row0, ROWS_PW)], idx_v)

    bufs = tuple(zip(rows_bufs, gsems, osems))

    def fire(rows, gsem, cp):
        pltpu.async_copy(
            tab_hbm.at[idx_v.at[2 * cp]], rows.at[pl.ds(0, X2_COLS)], gsem
        )
        pltpu.async_copy(
            tab_hbm.at[idx_v.at[2 * cp + 1]],
            rows.at[pl.ds(X2_COLS, X2_COLS)],
            gsem,
        )

    for b, (rows, gsem, _) in enumerate(bufs):
        fire(rows, gsem, b)

    def step(cc, carry):
        for b, (rows, gsem, osem) in enumerate(bufs):
            cp = NBUF * cc + b
            pltpu.make_async_copy(
                tab_hbm.at[pl.ds(0, 2 * X2_COLS)], rows, gsem
            ).wait()

            @pl.when(cc >= 1)
            def _():
                pltpu.make_async_copy(
                    outb.at[pl.ds(4 * b, 4)], out_hbm.at[pl.ds(0, 4)], osem
                ).wait()

            for s in range(4):
                row = 4 * b + s
                init = tuple(
                    rows[s * L, pl.ds(d * 16, 16)] for d in range(D // 16)
                )

                def _acc_body(j, acc):
                    return tuple(
                        a + rows[j, pl.ds(d * 16, 16)]
                        for d, a in enumerate(acc)
                    )

                acc = plsc.parallel_loop(
                    s * L + 1, s * L + L, unroll=7, carry=init
                )(_acc_body)
                for d in range(D // 16):
                    outb[row, pl.ds(d * 16, 16)] = acc[d]
            pltpu.async_copy(
                outb.at[pl.ds(4 * b, 4)], out_hbm.at[pl.ds(out0 + 4 * cp, 4)], osem
            )
            ncp = cp + NBUF

            @pl.when(ncp < CPW)
            def _():
                fire(rows, gsem, ncp)

        return carry

    lax.fori_loop(0, CPW // NBUF, step, 0)
    for b, (_, _, osem) in enumerate(bufs):
        pltpu.make_async_copy(
            outb.at[pl.ds(4 * b, 4)], out_hbm.at[pl.ds(0, 4)], osem
        ).wait()


def _make_pool(half):
    return functools.partial(
        pl.kernel,
        out_type=jax.ShapeDtypeStruct((BS, D), jnp.float32),
        mesh=plsc.VectorSubcoreMesh(core_axis_name="c", subcore_axis_name="s"),
        scratch_types=[
            pltpu.VMEM((ROWS_PW, X2_COLS), jnp.int32),
            tuple(
                pltpu.VMEM((2 * X2_COLS, D), jnp.float32) for _ in range(NBUF)
            ),
            pltpu.VMEM((4 * NBUF, D), jnp.float32),
            tuple(pltpu.SemaphoreType.DMA for _ in range(NBUF)),
            tuple(pltpu.SemaphoreType.DMA for _ in range(NBUF)),
        ],
    )(functools.partial(_pool_body, half))


_pools = [_make_pool(h) for h in range(NSPLIT)]


def _mlp_body(a_ref, w1_ref, b1_ref, w2_ref, b2_ref, w3_ref, b3_ref, o_ref):
    a = a_ref[...].astype(jnp.bfloat16)
    h1 = lax.dot_general(
        w1_ref[...], a, (((0,), (1,)), ((), ())),
        preferred_element_type=jnp.float32,
    ) + b1_ref[...]
    h1 = jnp.where(h1 > 0, h1, SLOPE * h1).astype(jnp.bfloat16)
    h2 = lax.dot_general(
        w2_ref[...], h1, (((0,), (0,)), ((), ())),
        preferred_element_type=jnp.float32,
    ) + b2_ref[...]
    h2 = jnp.where(h2 > 0, h2, SLOPE * h2).astype(jnp.bfloat16)
    lg = lax.dot_general(
        w3_ref[...], h2, (((0,), (0,)), ((), ())),
        preferred_element_type=jnp.float32,
    ) + b3_ref[...]
    m = jnp.max(lg, axis=0, keepdims=True)
    lse = jnp.log(jnp.sum(jnp.exp(lg - m), axis=0, keepdims=True))
    o_ref[...] = (lg - m) - lse


BT = 512


def _mlp_pass_body(a_ref, prev_ref, w1_ref, b1_ref, w2_ref, b2_ref, w3_ref,
                   b3_ref, o_ref):
    del prev_ref
    _mlp_body(a_ref, w1_ref, b1_ref, w2_ref, b2_ref, w3_ref, b3_ref, o_ref)


def _make_mlp(split):
    blk0 = split * (BS // BT)
    w_specs = [
        pl.BlockSpec((D, H1), lambda i: (0, 0)),
        pl.BlockSpec((H1, 1), lambda i: (0, 0)),
        pl.BlockSpec((H1, D), lambda i: (0, 0)),
        pl.BlockSpec((D, 1), lambda i: (0, 0)),
        pl.BlockSpec((D, C), lambda i: (0, 0)),
        pl.BlockSpec((C, 1), lambda i: (0, 0)),
    ]
    if split == 0:
        return pl.pallas_call(
            _mlp_body,
            grid=(BS // BT,),
            in_specs=[pl.BlockSpec((BT, D), lambda i: (i, 0))] + w_specs,
            out_specs=pl.BlockSpec((C, BT), lambda i: (0, i + blk0)),
            out_shape=jax.ShapeDtypeStruct((C, B), jnp.float32),
        )
    return pl.pallas_call(
        _mlp_pass_body,
        grid=(BS // BT,),
        in_specs=[
            pl.BlockSpec((BT, D), lambda i: (i, 0)),
            pl.BlockSpec((8, 128), lambda i: (0, 0)),
        ] + w_specs,
        out_specs=pl.BlockSpec((C, BT), lambda i: (0, i + blk0)),
        out_shape=jax.ShapeDtypeStruct((C, B), jnp.float32),
        input_output_aliases={1: 0},
    )


_mlps = [_make_mlp(sp) for sp in range(NSPLIT)]


def kernel(X_batch, table, W1, b1, W2, b2, W3, b3):
    xi = X_batch.astype(jnp.int32)
    xs = [
        lax.slice_in_dim(xi, sp * BS, (sp + 1) * BS).reshape(BS // 2, X2_COLS)
        for sp in range(NSPLIT)
    ]
    ws = (
        (W1 * (1.0 / L)).astype(jnp.bfloat16),
        b1.reshape(H1, 1),
        W2.astype(jnp.bfloat16),
        b2.reshape(D, 1),
        W3.astype(jnp.bfloat16),
        b3.reshape(C, 1),
    )
    pooled = [_pools[sp](xs[sp], table) for sp in range(NSPLIT)]
    out_t = _mlps[0](pooled[0], *ws)
    for sp in range(1, NSPLIT):
        out_t = _mlps[sp](pooled[sp], out_t, *ws)
    return out_t.T


# speedup vs baseline: 1.1157x; 1.1157x over previous
import functools

import jax
import jax.numpy as jnp
from jax import lax
from jax.experimental import pallas as pl
from jax.experimental.pallas import tpu as pltpu
from jax.experimental.pallas import tpu_sc as plsc

B, L, V, D, H1, C = 16384, 50, 100000, 128, 1024, 1000
SLOPE = 0.01

NC, NS = 2, 16
NW = NC * NS
NSPLIT = 2
BS = B // NSPLIT
SPW = BS // NW
X2_COLS = 2 * L
ROWS_PW = SPW // 2
NBUF = 4
CPW = ROWS_PW // 2


def _pool_body(half, x2_hbm, tab_hbm, out_hbm, idx_v, rows_bufs, outb, gsems,
               osems):
    del half
    wid = lax.axis_index("s") * NC + lax.axis_index("c")
    row0 = wid * ROWS_PW
    out0 = wid * SPW

    pltpu.sync_copy(x2_hbm.at[pl.ds(row0, ROWS_PW)], idx_v)

    bufs = tuple(zip(rows_bufs, gsems, osems))

    def fire(rows, gsem, cp):
        pltpu.async_copy(
            tab_hbm.at[idx_v.at[2 * cp]], rows.at[pl.ds(0, X2_COLS)], gsem
        )
        pltpu.async_copy(
            tab_hbm.at[idx_v.at[2 * cp + 1]],
            rows.at[pl.ds(X2_COLS, X2_COLS)],
            gsem,
        )

    for b, (rows, gsem, _) in enumerate(bufs):
        fire(rows, gsem, b)

    def step(cc, carry):
        for b, (rows, gsem, osem) in enumerate(bufs):
            cp = NBUF * cc + b
            pltpu.make_async_copy(
                tab_hbm.at[pl.ds(0, 2 * X2_COLS)], rows, gsem
            ).wait()

            @pl.when(cc >= 1)
            def _():
                pltpu.make_async_copy(
                    outb.at[pl.ds(4 * b, 4)], out_hbm.at[pl.ds(0, 4)], osem
                ).wait()

            for s in range(4):
                row = 4 * b + s
                init = tuple(
                    rows[s * L, pl.ds(d * 16, 16)] for d in range(D // 16)
                )

                def _acc_body(j, acc):
                    return tuple(
                        a + rows[j, pl.ds(d * 16, 16)]
                        for d, a in enumerate(acc)
                    )

                acc = plsc.parallel_loop(
                    s * L + 1, s * L + L, unroll=7, carry=init
                )(_acc_body)
                for d in range(D // 16):
                    outb[row, pl.ds(d * 16, 16)] = acc[d]
            pltpu.async_copy(
                outb.at[pl.ds(4 * b, 4)], out_hbm.at[pl.ds(out0 + 4 * cp, 4)], osem
            )
            ncp = cp + NBUF

            @pl.when(ncp < CPW)
            def _():
                fire(rows, gsem, ncp)

        return carry

    lax.fori_loop(0, CPW // NBUF, step, 0)
    for b, (_, _, osem) in enumerate(bufs):
        pltpu.make_async_copy(
            outb.at[pl.ds(4 * b, 4)], out_hbm.at[pl.ds(0, 4)], osem
        ).wait()


def _make_pool(half):
    return functools.partial(
        pl.kernel,
        out_type=jax.ShapeDtypeStruct((BS, D), jnp.float32),
        mesh=plsc.VectorSubcoreMesh(core_axis_name="c", subcore_axis_name="s"),
        scratch_types=[
            pltpu.VMEM((ROWS_PW, X2_COLS), jnp.int32),
            tuple(
                pltpu.VMEM((2 * X2_COLS, D), jnp.float32) for _ in range(NBUF)
            ),
            pltpu.VMEM((4 * NBUF, D), jnp.float32),
            tuple(pltpu.SemaphoreType.DMA for _ in range(NBUF)),
            tuple(pltpu.SemaphoreType.DMA for _ in range(NBUF)),
        ],
    )(functools.partial(_pool_body, half))


_pools = [_make_pool(h) for h in range(NSPLIT)]


def _mlp_body(a_ref, w1_ref, b1_ref, w2_ref, b2_ref, w3_ref, b3_ref, o_ref):
    a = a_ref[...].astype(jnp.bfloat16)
    h1 = lax.dot_general(
        w1_ref[...], a, (((0,), (1,)), ((), ())),
        preferred_element_type=jnp.float32,
    ) + b1_ref[...]
    h1 = jnp.where(h1 > 0, h1, SLOPE * h1).astype(jnp.bfloat16)
    h2 = lax.dot_general(
        w2_ref[...], h1, (((0,), (0,)), ((), ())),
        preferred_element_type=jnp.float32,
    ) + b2_ref[...]
    h2 = jnp.where(h2 > 0, h2, SLOPE * h2).astype(jnp.bfloat16)
    lg = lax.dot_general(
        w3_ref[...], h2, (((0,), (0,)), ((), ())),
        preferred_element_type=jnp.float32,
    ) + b3_ref[...]
    m = jnp.max(lg, axis=0, keepdims=True)
    lse = jnp.log(jnp.sum(jnp.exp(lg - m), axis=0, keepdims=True))
    o_ref[...] = (lg - m) - lse


BT = 512


def _mlp_pass_body(a_ref, prev_ref, w1_ref, b1_ref, w2_ref, b2_ref, w3_ref,
                   b3_ref, o_ref):
    del prev_ref
    _mlp_body(a_ref, w1_ref, b1_ref, w2_ref, b2_ref, w3_ref, b3_ref, o_ref)


def _make_mlp(split):
    blk0 = split * (BS // BT)
    w_specs = [
        pl.BlockSpec((D, H1), lambda i: (0, 0)),
        pl.BlockSpec((H1, 1), lambda i: (0, 0)),
        pl.BlockSpec((H1, D), lambda i: (0, 0)),
        pl.BlockSpec((D, 1), lambda i: (0, 0)),
        pl.BlockSpec((D, C), lambda i: (0, 0)),
        pl.BlockSpec((C, 1), lambda i: (0, 0)),
    ]
    if split == 0:
        return pl.pallas_call(
            _mlp_body,
            grid=(BS // BT,),
            in_specs=[pl.BlockSpec((BT, D), lambda i: (i, 0))] + w_specs,
            out_specs=pl.BlockSpec((C, BT), lambda i: (0, i + blk0)),
            out_shape=jax.ShapeDtypeStruct((C, B), jnp.float32),
        )
    return pl.pallas_call(
        _mlp_pass_body,
        grid=(BS // BT,),
        in_specs=[
            pl.BlockSpec((BT, D), lambda i: (i, 0)),
            pl.BlockSpec((8, 128), lambda i: (0, 0)),
        ] + w_specs,
        out_specs=pl.BlockSpec((C, BT), lambda i: (0, i + blk0)),
        out_shape=jax.ShapeDtypeStruct((C, B), jnp.float32),
        input_output_aliases={1: 0},
    )


_mlps = [_make_mlp(sp) for sp in range(NSPLIT)]


def kernel(X_batch, table, W1, b1, W2, b2, W3, b3):
    xi = X_batch.astype(jnp.int32)
    xs = [
        lax.slice_in_dim(xi, sp * BS, (sp + 1) * BS).reshape(BS // 2, X2_COLS)
        for sp in range(NSPLIT)
    ]
    ws = (
        (W1 * (1.0 / L)).astype(jnp.bfloat16),
        b1.reshape(H1, 1),
        W2.astype(jnp.bfloat16),
        b2.reshape(D, 1),
        W3.astype(jnp.bfloat16),
        b3.reshape(C, 1),
    )
    pooled = [_pools[sp](xs[sp], table) for sp in range(NSPLIT)]
    out_t = _mlps[0](pooled[0], *ws)
    for sp in range(1, NSPLIT):
        out_t = _mlps[sp](pooled[sp], out_t, *ws)
    return out_t.T


# speedup vs baseline: 1.1463x; 1.0274x over previous
import functools

import jax
import jax.numpy as jnp
from jax import lax
from jax.experimental import pallas as pl
from jax.experimental.pallas import tpu as pltpu
from jax.experimental.pallas import tpu_sc as plsc

B, L, V, D, H1, C = 16384, 50, 100000, 128, 1024, 1000
SLOPE = 0.01

NC, NS = 2, 16
NW = NC * NS
NSPLIT = 2
BS = B // NSPLIT
SPW = BS // NW
X2_COLS = 2 * L
ROWS_PW = SPW // 2
NBUF = 4


def _pool_body(half, x2_hbm, tab_hbm, out_hbm, idx_v, rows_bufs, outb, gsems,
               osems):
    del half
    wid = lax.axis_index("s") * NC + lax.axis_index("c")
    row0 = wid * ROWS_PW
    out0 = wid * SPW

    pltpu.sync_copy(x2_hbm.at[pl.ds(row0, ROWS_PW)], idx_v)

    bufs = tuple(zip(rows_bufs, gsems, osems))
    for b, (rows, gsem, _) in enumerate(bufs):
        pltpu.async_copy(tab_hbm.at[idx_v.at[b]], rows, gsem)

    def step(cc, carry):
        for b, (rows, gsem, osem) in enumerate(bufs):
            c = NBUF * cc + b
            pltpu.make_async_copy(tab_hbm.at[idx_v.at[0]], rows, gsem).wait()

            @pl.when(cc >= 1)
            def _():
                pltpu.make_async_copy(
                    outb.at[pl.ds(2 * b, 2)], out_hbm.at[pl.ds(0, 2)], osem
                ).wait()

            for s in range(2):
                row = 2 * b + s
                init = tuple(
                    rows[s * L, pl.ds(d * 16, 16)] for d in range(D // 16)
                )

                def _acc_body(j, acc):
                    return tuple(
                        a + rows[j, pl.ds(d * 16, 16)]
                        for d, a in enumerate(acc)
                    )

                acc = plsc.parallel_loop(
                    s * L + 1, s * L + L, unroll=7, carry=init
                )(_acc_body)
                for d in range(D // 16):
                    outb[row, pl.ds(d * 16, 16)] = acc[d]
            pltpu.async_copy(
                outb.at[pl.ds(2 * b, 2)], out_hbm.at[pl.ds(out0 + 2 * c, 2)], osem
            )
            nxt = c + NBUF

            @pl.when(nxt < ROWS_PW)
            def _():
                pltpu.async_copy(tab_hbm.at[idx_v.at[nxt]], rows, gsem)

        return carry

    lax.fori_loop(0, ROWS_PW // NBUF, step, 0)
    for b, (_, _, osem) in enumerate(bufs):
        pltpu.make_async_copy(
            outb.at[pl.ds(2 * b, 2)], out_hbm.at[pl.ds(0, 2)], osem
        ).wait()


def _make_pool(half):
    return functools.partial(
        pl.kernel,
        out_type=jax.ShapeDtypeStruct((BS, D), jnp.float32),
        mesh=plsc.VectorSubcoreMesh(core_axis_name="c", subcore_axis_name="s"),
        scratch_types=[
            pltpu.VMEM((ROWS_PW, X2_COLS), jnp.int32),
            tuple(pltpu.VMEM((X2_COLS, D), jnp.float32) for _ in range(NBUF)),
            pltpu.VMEM((2 * NBUF, D), jnp.float32),
            tuple(pltpu.SemaphoreType.DMA for _ in range(NBUF)),
            tuple(pltpu.SemaphoreType.DMA for _ in range(NBUF)),
        ],
    )(functools.partial(_pool_body, half))


_pools = [_make_pool(h) for h in range(NSPLIT)]


def _mlp_body(a_ref, w1_ref, b1_ref, w2_ref, b2_ref, w3_ref, b3_ref, o_ref):
    a = a_ref[...].astype(jnp.bfloat16)
    h1 = lax.dot_general(
        w1_ref[...], a, (((0,), (1,)), ((), ())),
        preferred_element_type=jnp.float32,
    ) + b1_ref[...]
    h1 = jnp.where(h1 > 0, h1, SLOPE * h1).astype(jnp.bfloat16)
    h2 = lax.dot_general(
        w2_ref[...], h1, (((0,), (0,)), ((), ())),
        preferred_element_type=jnp.float32,
    ) + b2_ref[...]
    h2 = jnp.where(h2 > 0, h2, SLOPE * h2).astype(jnp.bfloat16)
    lg = lax.dot_general(
        w3_ref[...], h2, (((0,), (0,)), ((), ())),
        preferred_element_type=jnp.float32,
    ) + b3_ref[...]
    m = jnp.max(lg, axis=0, keepdims=True)
    lse = jnp.log(jnp.sum(jnp.exp(lg - m), axis=0, keepdims=True))
    o_ref[...] = (lg - m) - lse


BT = 512


def _mlp_pass_body(a_ref, prev_ref, w1_ref, b1_ref, w2_ref, b2_ref, w3_ref,
                   b3_ref, o_ref):
    del prev_ref
    _mlp_body(a_ref, w1_ref, b1_ref, w2_ref, b2_ref, w3_ref, b3_ref, o_ref)


def _make_mlp(split):
    blk0 = split * (BS // BT)
    w_specs = [
        pl.BlockSpec((D, H1), lambda i: (0, 0)),
        pl.BlockSpec((H1, 1), lambda i: (0, 0)),
        pl.BlockSpec((H1, D), lambda i: (0, 0)),
        pl.BlockSpec((D, 1), lambda i: (0, 0)),
        pl.BlockSpec((D, C), lambda i: (0, 0)),
        pl.BlockSpec((C, 1), lambda i: (0, 0)),
    ]
    if split == 0:
        return pl.pallas_call(
            _mlp_body,
            grid=(BS // BT,),
            in_specs=[pl.BlockSpec((BT, D), lambda i: (i, 0))] + w_specs,
            out_specs=pl.BlockSpec((C, BT), lambda i: (0, i + blk0)),
            out_shape=jax.ShapeDtypeStruct((C, B), jnp.float32),
        )
    return pl.pallas_call(
        _mlp_pass_body,
        grid=(BS // BT,),
        in_specs=[
            pl.BlockSpec((BT, D), lambda i: (i, 0)),
            pl.BlockSpec((8, 128), lambda i: (0, 0)),
        ] + w_specs,
        out_specs=pl.BlockSpec((C, BT), lambda i: (0, i + blk0)),
        out_shape=jax.ShapeDtypeStruct((C, B), jnp.float32),
        input_output_aliases={1: 0},
    )


_mlps = [_make_mlp(sp) for sp in range(NSPLIT)]


def kernel(X_batch, table, W1, b1, W2, b2, W3, b3):
    xi = X_batch.astype(jnp.int32)
    xs = [
        lax.slice_in_dim(xi, sp * BS, (sp + 1) * BS).reshape(BS // 2, X2_COLS)
        for sp in range(NSPLIT)
    ]
    ws = (
        (W1 * (1.0 / L)).astype(jnp.bfloat16),
        b1.reshape(H1, 1),
        W2.astype(jnp.bfloat16),
        b2.reshape(D, 1),
        W3.astype(jnp.bfloat16),
        b3.reshape(C, 1),
    )
    pooled = [_pools[sp](xs[sp], table) for sp in range(NSPLIT)]
    out_t = _mlps[0](pooled[0], *ws)
    for sp in range(1, NSPLIT):
        out_t = _mlps[sp](pooled[sp], out_t, *ws)
    return out_t.T
